# baseline (speedup 1.0000x reference)
import jax
import jax.numpy as jnp
from jax import lax
from jax.experimental import pallas as pl
from jax.experimental.pallas import tpu as pltpu
from jax.experimental.pallas import tpu_sc as plsc

TOKENS = 4096
DMODEL = 2048
SIZE = 8
LANES = 16
ROWLEN = DMODEL * SIZE

NUM_CORES = 2
NUM_SUBCORES = 16
NUM_WORKERS = NUM_CORES * NUM_SUBCORES
TOK_PER_WORKER = TOKENS // NUM_WORKERS

CHUNK_TOKENS = 2
CHUNK_IN = CHUNK_TOKENS * ROWLEN
CHUNK_OUT = CHUNK_TOKENS * DMODEL
NUM_CHUNKS = TOK_PER_WORKER // CHUNK_TOKENS
NBUF = 2
GVECS = DMODEL // LANES


def _select_body(in_hbm, sel_hbm, out_hbm, sel_v, buf0, buf1, obuf0, obuf1,
                 in_sems, out_sems):
    bufs = [buf0, buf1]
    obufs = [obuf0, obuf1]
    cid = lax.axis_index("c")
    sid = lax.axis_index("s")
    wid = sid * NUM_CORES + cid

    pltpu.sync_copy(sel_hbm, sel_v)
    selv = sel_v[...]
    base_cols = selv + SIZE * lax.iota(jnp.int32, LANES)

    in_base = wid * TOK_PER_WORKER * ROWLEN
    out_base = wid * TOK_PER_WORKER * DMODEL

    def in_copy(i, b):
        return pltpu.make_async_copy(
            in_hbm.at[pl.ds(in_base + i * CHUNK_IN, CHUNK_IN)],
            bufs[b],
            in_sems.at[b],
        )

    def out_copy(i, b):
        return pltpu.make_async_copy(
            obufs[b],
            out_hbm.at[pl.ds(out_base + i * CHUNK_OUT, CHUNK_OUT)],
            out_sems.at[b],
        )

    def gather_chunk(b):
        for t in range(CHUNK_TOKENS):
            row = base_cols + t * ROWLEN
            for g in range(GVECS):
                v = plsc.load_gather(bufs[b], [row + g * (SIZE * LANES)])
                obufs[b][pl.ds(t * DMODEL + g * LANES, LANES)] = v

    for b in range(NBUF):
        in_copy(b, b).start()

    for b in range(NBUF):
        in_copy(b, b).wait()
        gather_chunk(b)
        out_copy(b, b).start()
        in_copy(b + NBUF, b).start()

    def pair_body(p, _):
        for b in range(NBUF):
            i = p * NBUF + b
            in_copy(i, b).wait()
            out_copy(i - NBUF, b).wait()
            gather_chunk(b)
            out_copy(i, b).start()
            in_copy(i + NBUF, b).start()
        return 0

    lax.fori_loop(1, NUM_CHUNKS // NBUF - 1, pair_body, 0)

    for b in range(NBUF):
        i = NUM_CHUNKS - NBUF + b
        in_copy(i, b).wait()
        out_copy(i - NBUF, b).wait()
        gather_chunk(b)
        out_copy(i, b).start()
    for b in range(NBUF):
        out_copy(NUM_CHUNKS - NBUF + b, b).wait()


@jax.jit
def _sc_select(input1d, sel16):
    mesh = plsc.VectorSubcoreMesh(core_axis_name="c", subcore_axis_name="s")
    return pl.kernel(
        _select_body,
        out_type=jax.ShapeDtypeStruct((TOKENS * DMODEL,), jnp.float32),
        mesh=mesh,
        scratch_types=[
            pltpu.VMEM((16,), jnp.int32),
            pltpu.VMEM((CHUNK_IN,), jnp.float32),
            pltpu.VMEM((CHUNK_IN,), jnp.float32),
            pltpu.VMEM((CHUNK_OUT,), jnp.float32),
            pltpu.VMEM((CHUNK_OUT,), jnp.float32),
            pltpu.SemaphoreType.DMA((NBUF,)),
            pltpu.SemaphoreType.DMA((NBUF,)),
        ],
        compiler_params=pltpu.CompilerParams(needs_layout_passes=False),
    )(input1d, sel16)


def kernel(input, total_loss, pre_sels, weight):
    del total_loss, weight
    input1d = input.reshape(TOKENS * ROWLEN)
    sel16 = jnp.broadcast_to(pre_sels.astype(jnp.int32), (16,))
    return _sc_select(input1d, sel16).reshape(TOKENS, DMODEL)
